# baseline (speedup 1.0000x reference)
import functools

import jax
import jax.numpy as jnp
from jax import lax
from jax.experimental import pallas as pl
from jax.experimental.pallas import tpu as pltpu
from jax.experimental.pallas import tpu_sc as plsc


def _make_sc_lin(B, D, NC, NS, L):
    NW = NC * NS
    bw = B // NW

    mesh = plsc.VectorSubcoreMesh(core_axis_name="c", subcore_axis_name="s")

    @functools.partial(
        pl.kernel,
        mesh=mesh,
        compiler_params=pltpu.CompilerParams(
            use_tc_tiling_on_sc=False,
            needs_layout_passes=False,
        ),
        out_type=(
            jax.ShapeDtypeStruct((B, D), jnp.float32),
            jax.ShapeDtypeStruct((B,), jnp.float32),
        ),
        scratch_types=[
            pltpu.VMEM((bw,), jnp.int32),
            pltpu.VMEM((bw,), jnp.int32),
            pltpu.VMEM((bw, D), jnp.float32),
            pltpu.VMEM((bw,), jnp.float32),
            pltpu.VMEM((bw,), jnp.float32),
            pltpu.VMEM((bw,), jnp.float32),
            pltpu.SemaphoreType.DMA,
            pltpu.SemaphoreType.DMA,
            pltpu.SemaphoreType.DMA,
        ],
    )
    def sc_a(ls_hbm, rs_hbm, rw_hbm, lb_hbm, rb_hbm,
             rwrows_hbm, c_hbm,
             ls_v, rs_v, rw_v, lb_v, rb_v, c_v, sem1, sem2, sem3):
        wid = lax.axis_index("s") * NC + lax.axis_index("c")
        base = wid * bw
        pltpu.sync_copy(ls_hbm.at[pl.ds(base, bw)], ls_v)
        pltpu.sync_copy(rs_hbm.at[pl.ds(base, bw)], rs_v)
        h1 = pltpu.async_copy(rw_hbm.at[rs_v], rw_v, sem1)
        h2 = pltpu.async_copy(lb_hbm.at[ls_v], lb_v, sem2)
        h3 = pltpu.async_copy(rb_hbm.at[rs_v], rb_v, sem3)
        h2.wait()
        h3.wait()
        for g in range(bw // L):
            c_v[pl.ds(g * L, L)] = lb_v[pl.ds(g * L, L)] + rb_v[pl.ds(g * L, L)]
        pltpu.sync_copy(c_v, c_hbm.at[pl.ds(base, bw)])
        h1.wait()
        pltpu.sync_copy(rw_v, rwrows_hbm.at[pl.ds(base, bw)])

    return sc_a


def _make_sc_til(B, D, NC, NS, L):
    NW = NC * NS
    bw = B // NW
    RING = 8

    mesh = plsc.VectorSubcoreMesh(core_axis_name="c", subcore_axis_name="s")

    @functools.partial(
        pl.kernel,
        mesh=mesh,
        compiler_params=pltpu.CompilerParams(
            use_tc_tiling_on_sc=True,
            needs_layout_passes=False,
        ),
        out_type=jax.ShapeDtypeStruct((B, D), jnp.float32),
        scratch_types=[
            pltpu.VMEM((bw,), jnp.int32),
            pltpu.VMEM((bw, D), jnp.float32),
        ] + [pltpu.VMEM((D, 128), jnp.float32) for _ in range(8)] + [
            pltpu.SemaphoreType.DMA for _ in range(8)
        ],
    )
    def sc_b(ls_hbm, lwT_hbm, lwrows_hbm,
             ls_v, lw_v,
             t0, t1, t2, t3, t4, t5, t6, t7,
             s0, s1, s2, s3, s4, s5, s6, s7):
        tbufs = [t0, t1, t2, t3, t4, t5, t6, t7]
        sems = [s0, s1, s2, s3, s4, s5, s6, s7]
        wid = lax.axis_index("s") * NC + lax.axis_index("c")
        base = wid * bw
        pltpu.sync_copy(ls_hbm.at[pl.ds(base, bw)], ls_v)

        lanes = lax.iota(jnp.int32, L)
        lane_eq = [lanes == j for j in range(L)]
        ls_chunks = [ls_v[pl.ds(g * L, L)] for g in range(bw // L)]

        def col_scalar(j):
            masked = jnp.where(lane_eq[j % L], ls_chunks[j // L],
                               jnp.int32(-2147483648))
            return jnp.max(masked)

        def fire(j):
            col = col_scalar(j)
            tcol = pl.multiple_of((col // 128) * 128, 128)
            return pltpu.async_copy(
                lwT_hbm.at[:, pl.ds(tcol, 128)], tbufs[j % RING],
                sems[j % RING])

        handles = {}
        for j in range(RING):
            handles[j] = fire(j)

        d0s = [lanes + (k * L) for k in range(D // L)]
        for j in range(bw):
            handles.pop(j).wait()
            buf = tbufs[j % RING]
            qv = jnp.full((L,), col_scalar(j) % 128, jnp.int32)
            for k in range(D // L):
                lw_v[j, pl.ds(k * L, L)] = plsc.load_gather(buf, [d0s[k], qv])
            if j + RING < bw:
                handles[j + RING] = fire(j + RING)
        pltpu.sync_copy(lw_v, lwrows_hbm.at[pl.ds(base, bw), :])

    return sc_b


def _fused_body(c_ref, lw_ref, rw_ref, o_ref):
    prod = lw_ref[...] * rw_ref[...]
    dot = jnp.sum(prod, axis=1)
    o_ref[...] = c_ref[...] + dot[None, :]


def kernel(ls, rs, LW, Lb, RW, Rb):
    B = ls.shape[0]
    N, D = LW.shape
    M = RW.shape[0]
    info = plsc.get_sparse_core_info()
    NC, NS, L = info.num_cores, info.num_subcores, info.num_lanes
    ls32 = ls.astype(jnp.int32)
    rs32 = rs.astype(jnp.int32)

    sc_b = _make_sc_til(B, D, NC, NS, L)
    lw_rows = sc_b(ls32, LW.T)

    sc_a = _make_sc_lin(B, D, NC, NS, L)
    rw_rows, c = sc_a(ls32, rs32, RW, Lb.reshape(N), Rb.reshape(M))

    BN = 256
    out = pl.pallas_call(
        _fused_body,
        grid=(B // BN,),
        in_specs=[
            pl.BlockSpec((B, 1), lambda i: (0, 0)),
            pl.BlockSpec((BN, D), lambda i: (i, 0)),
            pl.BlockSpec((BN, D), lambda i: (i, 0)),
        ],
        out_specs=pl.BlockSpec((B, BN), lambda i: (0, i)),
        out_shape=jax.ShapeDtypeStruct((B, B), jnp.float32),
    )(c.reshape(B, 1), lw_rows, rw_rows)
    return out


# speedup vs baseline: 1.3665x; 1.3665x over previous
import functools

import jax
import jax.numpy as jnp
from jax import lax
from jax.experimental import pallas as pl
from jax.experimental.pallas import tpu as pltpu
from jax.experimental.pallas import tpu_sc as plsc


def _make_sc_lin(B, D, NC, NS, L):
    NW = NC * NS
    bw = B // NW

    mesh = plsc.VectorSubcoreMesh(core_axis_name="c", subcore_axis_name="s")

    @functools.partial(
        pl.kernel,
        mesh=mesh,
        compiler_params=pltpu.CompilerParams(
            use_tc_tiling_on_sc=False,
            needs_layout_passes=False,
        ),
        out_type=(
            jax.ShapeDtypeStruct((B, D), jnp.float32),
            jax.ShapeDtypeStruct((B,), jnp.float32),
        ),
        scratch_types=[
            pltpu.VMEM((bw,), jnp.int32),
            pltpu.VMEM((bw,), jnp.int32),
            pltpu.VMEM((bw, D), jnp.float32),
            pltpu.VMEM((bw,), jnp.float32),
            pltpu.VMEM((bw,), jnp.float32),
            pltpu.VMEM((bw,), jnp.float32),
            pltpu.SemaphoreType.DMA,
            pltpu.SemaphoreType.DMA,
            pltpu.SemaphoreType.DMA,
        ],
    )
    def sc_a(ls_hbm, rs_hbm, rw_hbm, lb_hbm, rb_hbm,
             rwrows_hbm, c_hbm,
             ls_v, rs_v, rw_v, lb_v, rb_v, c_v, sem1, sem2, sem3):
        wid = lax.axis_index("s") * NC + lax.axis_index("c")
        base = wid * bw
        pltpu.sync_copy(ls_hbm.at[pl.ds(base, bw)], ls_v)
        pltpu.sync_copy(rs_hbm.at[pl.ds(base, bw)], rs_v)
        h1 = pltpu.async_copy(rw_hbm.at[rs_v], rw_v, sem1)
        h2 = pltpu.async_copy(lb_hbm.at[ls_v], lb_v, sem2)
        h3 = pltpu.async_copy(rb_hbm.at[rs_v], rb_v, sem3)
        h2.wait()
        h3.wait()
        for g in range(bw // L):
            c_v[pl.ds(g * L, L)] = lb_v[pl.ds(g * L, L)] + rb_v[pl.ds(g * L, L)]
        pltpu.sync_copy(c_v, c_hbm.at[pl.ds(base, bw)])
        h1.wait()
        pltpu.sync_copy(rw_v, rwrows_hbm.at[pl.ds(base, bw)])

    return sc_a


def _make_sc_til(B, D, NC, NS, L):
    NW = NC * NS
    bw = B // NW
    RING = 8

    mesh = plsc.VectorSubcoreMesh(core_axis_name="c", subcore_axis_name="s")

    @functools.partial(
        pl.kernel,
        mesh=mesh,
        compiler_params=pltpu.CompilerParams(
            use_tc_tiling_on_sc=True,
            needs_layout_passes=False,
        ),
        out_type=jax.ShapeDtypeStruct((B, D), jnp.float32),
        scratch_types=[
            pltpu.VMEM((bw,), jnp.int32),
            pltpu.VMEM((bw, D), jnp.float32),
        ] + [pltpu.VMEM((D, 128), jnp.float32) for _ in range(8)] + [
            pltpu.SemaphoreType.DMA for _ in range(8)
        ],
    )
    def sc_b(ls_hbm, lwT_hbm, lwrows_hbm,
             ls_v, lw_v,
             t0, t1, t2, t3, t4, t5, t6, t7,
             s0, s1, s2, s3, s4, s5, s6, s7):
        tbufs = [t0, t1, t2, t3, t4, t5, t6, t7]
        sems = [s0, s1, s2, s3, s4, s5, s6, s7]
        wid = lax.axis_index("s") * NC + lax.axis_index("c")
        base = wid * bw
        pltpu.sync_copy(ls_hbm.at[pl.ds(base, bw)], ls_v)

        lanes = lax.iota(jnp.int32, L)
        lane_eq = [lanes == j for j in range(L)]
        ls_chunks = [ls_v[pl.ds(g * L, L)] for g in range(bw // L)]

        def col_scalar(j):
            masked = jnp.where(lane_eq[j % L], ls_chunks[j // L],
                               jnp.int32(-2147483648))
            return jnp.max(masked)

        def fire(j):
            col = col_scalar(j)
            tcol = pl.multiple_of((col // 128) * 128, 128)
            return pltpu.async_copy(
                lwT_hbm.at[:, pl.ds(tcol, 128)], tbufs[j % RING],
                sems[j % RING])

        handles = {}
        for j in range(RING):
            handles[j] = fire(j)

        d0s = [lanes + (k * L) for k in range(D // L)]
        for j in range(bw):
            handles.pop(j).wait()
            buf = tbufs[j % RING]
            qv = jnp.full((L,), col_scalar(j) % 128, jnp.int32)
            for k in range(D // L):
                lw_v[j, pl.ds(k * L, L)] = plsc.load_gather(buf, [d0s[k], qv])
            if j + RING < bw:
                handles[j + RING] = fire(j + RING)
        pltpu.sync_copy(lw_v, lwrows_hbm.at[pl.ds(base, bw), :])

    return sc_b


def _fused_body(c_ref, lw_ref, rw_ref, o_ref):
    prod = lw_ref[...] * rw_ref[...]
    dot = jnp.sum(prod, axis=1)
    o_ref[...] = c_ref[...] + dot[None, :]


def kernel(ls, rs, LW, Lb, RW, Rb):
    B = ls.shape[0]
    N, D = LW.shape
    M = RW.shape[0]
    info = plsc.get_sparse_core_info()
    NC, NS, L = info.num_cores, info.num_subcores, info.num_lanes
    ls32 = ls.astype(jnp.int32)
    rs32 = rs.astype(jnp.int32)

    sc_b = _make_sc_til(B, D, NC, NS, L)
    lw_rows = sc_b(ls32, LW.T)

    rs32_b, _ = lax.optimization_barrier((rs32, lw_rows))
    sc_a = _make_sc_lin(B, D, NC, NS, L)
    rw_rows, c = sc_a(ls32, rs32_b, RW, Lb.reshape(N), Rb.reshape(M))

    BN = 256
    out = pl.pallas_call(
        _fused_body,
        grid=(B // BN,),
        in_specs=[
            pl.BlockSpec((B, 1), lambda i: (0, 0)),
            pl.BlockSpec((BN, D), lambda i: (i, 0)),
            pl.BlockSpec((BN, D), lambda i: (i, 0)),
        ],
        out_specs=pl.BlockSpec((B, BN), lambda i: (0, i)),
        out_shape=jax.ShapeDtypeStruct((B, B), jnp.float32),
    )(c.reshape(B, 1), lw_rows, rw_rows)
    return out


# speedup vs baseline: 1.3982x; 1.0232x over previous
import functools

import jax
import jax.numpy as jnp
from jax import lax
from jax.experimental import pallas as pl
from jax.experimental.pallas import tpu as pltpu
from jax.experimental.pallas import tpu_sc as plsc


def _make_sc_lin(B, D, NC, NS, L):
    NW = NC * NS
    bw = B // NW

    mesh = plsc.VectorSubcoreMesh(core_axis_name="c", subcore_axis_name="s")

    @functools.partial(
        pl.kernel,
        mesh=mesh,
        compiler_params=pltpu.CompilerParams(
            use_tc_tiling_on_sc=False,
            needs_layout_passes=False,
        ),
        out_type=(
            jax.ShapeDtypeStruct((B, 128), jnp.float32),
            jax.ShapeDtypeStruct((B,), jnp.float32),
        ),
        scratch_types=[
            pltpu.VMEM((bw,), jnp.int32),
            pltpu.VMEM((bw,), jnp.int32),
            pltpu.VMEM((bw, D), jnp.float32),
            pltpu.VMEM((bw,), jnp.float32),
            pltpu.VMEM((bw,), jnp.float32),
            pltpu.VMEM((bw,), jnp.float32),
            pltpu.SemaphoreType.DMA,
            pltpu.SemaphoreType.DMA,
            pltpu.SemaphoreType.DMA,
        ],
    )
    def sc_a(ls_hbm, rs_hbm, rw_hbm, lb_hbm, rb_hbm,
             rwrows_hbm, c_hbm,
             ls_v, rs_v, rw_v, lb_v, rb_v, c_v, sem1, sem2, sem3):
        wid = lax.axis_index("s") * NC + lax.axis_index("c")
        base = wid * bw
        pltpu.sync_copy(ls_hbm.at[pl.ds(base, bw)], ls_v)
        pltpu.sync_copy(rs_hbm.at[pl.ds(base, bw)], rs_v)
        h1 = pltpu.async_copy(rw_hbm.at[rs_v], rw_v, sem1)
        h2 = pltpu.async_copy(lb_hbm.at[ls_v], lb_v, sem2)
        h3 = pltpu.async_copy(rb_hbm.at[rs_v], rb_v, sem3)
        h2.wait()
        h3.wait()
        for g in range(bw // L):
            c_v[pl.ds(g * L, L)] = lb_v[pl.ds(g * L, L)] + rb_v[pl.ds(g * L, L)]
        pltpu.sync_copy(c_v, c_hbm.at[pl.ds(base, bw)])
        h1.wait()
        pltpu.sync_copy(rw_v, rwrows_hbm.at[pl.ds(base, bw), pl.ds(0, D)])

    return sc_a


def _make_sc_til(B, D, NC, NS, L):
    NW = NC * NS
    bw = B // NW
    RING = 8

    mesh = plsc.VectorSubcoreMesh(core_axis_name="c", subcore_axis_name="s")

    @functools.partial(
        pl.kernel,
        mesh=mesh,
        compiler_params=pltpu.CompilerParams(
            use_tc_tiling_on_sc=True,
            needs_layout_passes=False,
        ),
        out_type=jax.ShapeDtypeStruct((B, D), jnp.float32),
        scratch_types=[
            pltpu.VMEM((bw,), jnp.int32),
            pltpu.VMEM((bw, D), jnp.float32),
        ] + [pltpu.VMEM((D, 128), jnp.float32) for _ in range(8)] + [
            pltpu.SemaphoreType.DMA for _ in range(8)
        ],
    )
    def sc_b(ls_hbm, lwT_hbm, lwrows_hbm,
             ls_v, lw_v,
             t0, t1, t2, t3, t4, t5, t6, t7,
             s0, s1, s2, s3, s4, s5, s6, s7):
        tbufs = [t0, t1, t2, t3, t4, t5, t6, t7]
        sems = [s0, s1, s2, s3, s4, s5, s6, s7]
        wid = lax.axis_index("s") * NC + lax.axis_index("c")
        base = wid * bw
        pltpu.sync_copy(ls_hbm.at[pl.ds(base, bw)], ls_v)

        lanes = lax.iota(jnp.int32, L)
        lane_eq = [lanes == j for j in range(L)]
        ls_chunks = [ls_v[pl.ds(g * L, L)] for g in range(bw // L)]

        def col_scalar(j):
            masked = jnp.where(lane_eq[j % L], ls_chunks[j // L],
                               jnp.int32(-2147483648))
            return jnp.max(masked)

        def fire(j):
            col = col_scalar(j)
            tcol = pl.multiple_of((col // 128) * 128, 128)
            return pltpu.async_copy(
                lwT_hbm.at[:, pl.ds(tcol, 128)], tbufs[j % RING],
                sems[j % RING])

        handles = {}
        for j in range(RING):
            handles[j] = fire(j)

        d0s = [lanes + (k * L) for k in range(D // L)]
        for j in range(bw):
            handles.pop(j).wait()
            buf = tbufs[j % RING]
            qv = jnp.full((L,), col_scalar(j) % 128, jnp.int32)
            for k in range(D // L):
                lw_v[j, pl.ds(k * L, L)] = plsc.load_gather(buf, [d0s[k], qv])
            if j + RING < bw:
                handles[j + RING] = fire(j + RING)
        pltpu.sync_copy(lw_v, lwrows_hbm.at[pl.ds(base, bw), :])

    return sc_b


def _fused_body(c_ref, lw_ref, rw_ref, o_ref):
    d = lw_ref.shape[1]
    prod = lw_ref[...] * rw_ref[:, :d]
    dot = jnp.sum(prod, axis=1)
    o_ref[...] = c_ref[...] + dot[None, :]


def kernel(ls, rs, LW, Lb, RW, Rb):
    B = ls.shape[0]
    N, D = LW.shape
    M = RW.shape[0]
    info = plsc.get_sparse_core_info()
    NC, NS, L = info.num_cores, info.num_subcores, info.num_lanes
    ls32 = ls.astype(jnp.int32)
    rs32 = rs.astype(jnp.int32)

    sc_b = _make_sc_til(B, D, NC, NS, L)
    lw_rows = sc_b(ls32, LW.T)

    rs32_b, _ = lax.optimization_barrier((rs32, lw_rows))
    sc_a = _make_sc_lin(B, D, NC, NS, L)
    rw_rows, c = sc_a(ls32, rs32_b, RW, Lb.reshape(N), Rb.reshape(M))

    BN = 256
    out = pl.pallas_call(
        _fused_body,
        grid=(B // BN,),
        in_specs=[
            pl.BlockSpec((B, 1), lambda i: (0, 0)),
            pl.BlockSpec((BN, D), lambda i: (i, 0)),
            pl.BlockSpec((BN, 128), lambda i: (i, 0)),
        ],
        out_specs=pl.BlockSpec((B, BN), lambda i: (0, i)),
        out_shape=jax.ShapeDtypeStruct((B, B), jnp.float32),
    )(c.reshape(B, 1), lw_rows, rw_rows)
    return out


# speedup vs baseline: 1.4450x; 1.0334x over previous
import functools

import jax
import jax.numpy as jnp
from jax import lax
from jax.experimental import pallas as pl
from jax.experimental.pallas import tpu as pltpu
from jax.experimental.pallas import tpu_sc as plsc


def _make_sc_lin(B, D, NC, NS, L):
    NW = NC * NS
    bw = B // NW

    mesh = plsc.VectorSubcoreMesh(core_axis_name="c", subcore_axis_name="s")

    @functools.partial(
        pl.kernel,
        mesh=mesh,
        compiler_params=pltpu.CompilerParams(
            use_tc_tiling_on_sc=False,
            needs_layout_passes=False,
        ),
        out_type=(
            jax.ShapeDtypeStruct((B, 128), jnp.float32),
            jax.ShapeDtypeStruct((B,), jnp.float32),
        ),
        scratch_types=[
            pltpu.VMEM((bw,), jnp.int32),
            pltpu.VMEM((bw,), jnp.int32),
            pltpu.VMEM((bw, D), jnp.float32),
            pltpu.VMEM((bw,), jnp.float32),
            pltpu.VMEM((bw,), jnp.float32),
            pltpu.VMEM((bw,), jnp.float32),
            pltpu.SemaphoreType.DMA,
            pltpu.SemaphoreType.DMA,
            pltpu.SemaphoreType.DMA,
        ],
    )
    def sc_a(ls_hbm, rs_hbm, rw_hbm, lb_hbm, rb_hbm,
             rwrows_hbm, c_hbm,
             ls_v, rs_v, rw_v, lb_v, rb_v, c_v, sem1, sem2, sem3):
        wid = lax.axis_index("s") * NC + lax.axis_index("c")
        base = wid * bw
        pltpu.sync_copy(ls_hbm.at[pl.ds(base, bw)], ls_v)
        pltpu.sync_copy(rs_hbm.at[pl.ds(base, bw)], rs_v)
        h1 = pltpu.async_copy(rw_hbm.at[rs_v], rw_v, sem1)
        h2 = pltpu.async_copy(lb_hbm.at[ls_v], lb_v, sem2)
        h3 = pltpu.async_copy(rb_hbm.at[rs_v], rb_v, sem3)
        h2.wait()
        h3.wait()
        for g in range(bw // L):
            c_v[pl.ds(g * L, L)] = lb_v[pl.ds(g * L, L)] + rb_v[pl.ds(g * L, L)]
        pltpu.sync_copy(c_v, c_hbm.at[pl.ds(base, bw)])
        h1.wait()
        pltpu.sync_copy(rw_v, rwrows_hbm.at[pl.ds(base, bw), pl.ds(0, D)])

    return sc_a


def _make_sc_til(B, D, NC, NS, L):
    NW = NC * NS
    bw = B // NW
    RING = 8

    mesh = plsc.VectorSubcoreMesh(core_axis_name="c", subcore_axis_name="s")

    @functools.partial(
        pl.kernel,
        mesh=mesh,
        compiler_params=pltpu.CompilerParams(
            use_tc_tiling_on_sc=True,
            needs_layout_passes=False,
        ),
        out_type=jax.ShapeDtypeStruct((B, D), jnp.float32),
        scratch_types=[
            pltpu.VMEM((bw,), jnp.int32),
            pltpu.VMEM((bw, D), jnp.float32),
        ] + [pltpu.VMEM((D, 128), jnp.float32) for _ in range(8)] + [
            pltpu.SemaphoreType.DMA for _ in range(8)
        ],
    )
    def sc_b(ls_hbm, lwT_hbm, lwrows_hbm,
             ls_v, lw_v,
             t0, t1, t2, t3, t4, t5, t6, t7,
             s0, s1, s2, s3, s4, s5, s6, s7):
        tbufs = [t0, t1, t2, t3, t4, t5, t6, t7]
        sems = [s0, s1, s2, s3, s4, s5, s6, s7]
        wid = lax.axis_index("s") * NC + lax.axis_index("c")
        base = wid * bw
        pltpu.sync_copy(ls_hbm.at[pl.ds(base, bw)], ls_v)

        lanes = lax.iota(jnp.int32, L)
        lane_eq = [lanes == j for j in range(L)]
        ls_chunks = [ls_v[pl.ds(g * L, L)] for g in range(bw // L)]

        def col_scalar(j):
            masked = jnp.where(lane_eq[j % L], ls_chunks[j // L],
                               jnp.int32(-2147483648))
            return jnp.max(masked)

        def fire(j):
            col = col_scalar(j)
            tcol = pl.multiple_of((col // 128) * 128, 128)
            return pltpu.async_copy(
                lwT_hbm.at[:, pl.ds(tcol, 128)], tbufs[j % RING],
                sems[j % RING])

        handles = {}
        for j in range(RING):
            handles[j] = fire(j)

        d0s = [lanes + (k * L) for k in range(D // L)]
        for j in range(bw):
            handles.pop(j).wait()
            buf = tbufs[j % RING]
            qv = jnp.full((L,), col_scalar(j) % 128, jnp.int32)
            for k in range(D // L):
                lw_v[j, pl.ds(k * L, L)] = plsc.load_gather(buf, [d0s[k], qv])
            if j + RING < bw:
                handles[j + RING] = fire(j + RING)
        pltpu.sync_copy(lw_v, lwrows_hbm.at[pl.ds(base, bw), :])

    return sc_b


def _fused_body(c_ref, lw_ref, rw_ref, o_ref):
    d = lw_ref.shape[1]
    prod = lw_ref[...] * rw_ref[:, :d]
    dot = jnp.sum(prod, axis=1)
    o_ref[...] = c_ref[...] + dot[None, :]


def kernel(ls, rs, LW, Lb, RW, Rb):
    B = ls.shape[0]
    N, D = LW.shape
    M = RW.shape[0]
    info = plsc.get_sparse_core_info()
    NC, NS, L = info.num_cores, info.num_subcores, info.num_lanes
    ls32 = ls.astype(jnp.int32)
    rs32 = rs.astype(jnp.int32)

    sc_b = _make_sc_til(B, D, NC, NS, L)
    lw_rows = sc_b(ls32, LW.T)

    rs32_b, _ = lax.optimization_barrier((rs32, lw_rows))
    sc_a = _make_sc_lin(B, D, NC, NS, L)
    rw_rows, c = sc_a(ls32, rs32_b, RW, Lb.reshape(N), Rb.reshape(M))

    BN = 512
    out = pl.pallas_call(
        _fused_body,
        grid=(B // BN,),
        in_specs=[
            pl.BlockSpec((B, 1), lambda i: (0, 0)),
            pl.BlockSpec((BN, D), lambda i: (i, 0)),
            pl.BlockSpec((BN, 128), lambda i: (i, 0)),
        ],
        out_specs=pl.BlockSpec((B, BN), lambda i: (0, i)),
        out_shape=jax.ShapeDtypeStruct((B, B), jnp.float32),
    )(c.reshape(B, 1), lw_rows, rw_rows)
    return out
